# Initial kernel scaffold; baseline (speedup 1.0000x reference)
import jax
import jax.numpy as jnp
from jax.experimental import pallas as pl


def kernel(logit, target, base_m_list, class_margin_weights):
    raise NotImplementedError("write your pallas kernel here")


# speedup vs baseline: 4.2595x; 4.2595x over previous
import jax
import jax.numpy as jnp
from jax.experimental import pallas as pl
from jax.experimental.pallas import tpu as pltpu

_NUM_CLASSES = 100
_S = 30.0
_BLK = 2048


def _ldam_body(logit_ref, tgt_ref, bm_ref, w_ref, out_ref):
    i = pl.program_id(0)
    n = pl.num_programs(0)
    x = logit_ref[...]
    t = tgt_ref[...]
    m_vec = bm_ref[...] * jax.nn.sigmoid(w_ref[...])

    col = jax.lax.broadcasted_iota(jnp.int32, x.shape, 1)
    onehot = col == t
    picked = jnp.sum(jnp.where(onehot, x, 0.0), axis=1, keepdims=True)
    m_row = jnp.sum(jnp.where(onehot, m_vec, 0.0), axis=1, keepdims=True)
    adj = picked - _S * m_row

    mx = jnp.max(x, axis=1, keepdims=True)
    z = jnp.sum(jnp.exp(x - mx), axis=1, keepdims=True)
    zp = z - jnp.exp(picked - mx) + jnp.exp(adj - mx)
    loss = mx + jnp.log(zp) - adj
    s = jnp.sum(loss, axis=0, keepdims=True)

    @pl.when(i == 0)
    def _():
        out_ref[...] = jnp.zeros_like(out_ref)

    out_ref[...] += s

    @pl.when(i == n - 1)
    def _():
        out_ref[...] = out_ref[...] / (n * _BLK)


def kernel(logit, target, base_m_list, class_margin_weights):
    b, c = logit.shape
    tgt2 = target.reshape(b, 1)
    bm2 = base_m_list.reshape(1, c)
    w2 = class_margin_weights.reshape(1, c)
    grid = b // _BLK
    out = pl.pallas_call(
        _ldam_body,
        grid=(grid,),
        in_specs=[
            pl.BlockSpec((_BLK, c), lambda i: (i, 0)),
            pl.BlockSpec((_BLK, 1), lambda i: (i, 0)),
            pl.BlockSpec((1, c), lambda i: (0, 0)),
            pl.BlockSpec((1, c), lambda i: (0, 0)),
        ],
        out_specs=pl.BlockSpec((1, 1), lambda i: (0, 0)),
        out_shape=jax.ShapeDtypeStruct((1, 1), jnp.float32),
    )(logit, tgt2, bm2, w2)
    return out[0, 0]


# speedup vs baseline: 11.2573x; 2.6428x over previous
import jax
import jax.numpy as jnp
from jax.experimental import pallas as pl
from jax.experimental.pallas import tpu as pltpu

_NUM_CLASSES = 100
_S = 30.0
_BLKC = 2048


def _ldam_body(logit_ref, tgt_ref, bm_ref, w_ref, out_ref):
    i = pl.program_id(0)
    n = pl.num_programs(0)
    x = logit_ref[...]
    t = tgt_ref[...]
    m_vec = bm_ref[...] * jax.nn.sigmoid(w_ref[...])

    cls = jax.lax.broadcasted_iota(jnp.int32, x.shape, 0)
    onehot = cls == t
    picked = jnp.sum(jnp.where(onehot, x, 0.0), axis=0, keepdims=True)
    m_col = jnp.sum(jnp.where(onehot, m_vec, 0.0), axis=0, keepdims=True)
    adj = picked - _S * m_col

    mx = jnp.max(x, axis=0, keepdims=True)
    z = jnp.sum(jnp.exp(x - mx), axis=0, keepdims=True)
    zp = z - jnp.exp(picked - mx) + jnp.exp(adj - mx)
    loss = mx + jnp.log(zp) - adj
    s = jnp.sum(loss, axis=1, keepdims=True)

    @pl.when(i == 0)
    def _():
        out_ref[...] = jnp.zeros_like(out_ref)

    out_ref[...] += s

    @pl.when(i == n - 1)
    def _():
        out_ref[...] = out_ref[...] / (n * _BLKC)


def kernel(logit, target, base_m_list, class_margin_weights):
    b, c = logit.shape
    xt = logit.T
    tgt2 = target.reshape(1, b)
    bm2 = base_m_list.reshape(c, 1)
    w2 = class_margin_weights.reshape(c, 1)
    grid = b // _BLKC
    out = pl.pallas_call(
        _ldam_body,
        grid=(grid,),
        in_specs=[
            pl.BlockSpec((c, _BLKC), lambda i: (0, i)),
            pl.BlockSpec((1, _BLKC), lambda i: (0, i)),
            pl.BlockSpec((c, 1), lambda i: (0, 0)),
            pl.BlockSpec((c, 1), lambda i: (0, 0)),
        ],
        out_specs=pl.BlockSpec((1, 1), lambda i: (0, 0)),
        out_shape=jax.ShapeDtypeStruct((1, 1), jnp.float32),
    )(xt, tgt2, bm2, w2)
    return out[0, 0]
